# Initial kernel scaffold; baseline (speedup 1.0000x reference)
import jax
import jax.numpy as jnp
from jax.experimental import pallas as pl


def kernel(inputs, table):
    raise NotImplementedError("write your pallas kernel here")


# speedup vs baseline: 1.5659x; 1.5659x over previous
import functools

import jax
import jax.numpy as jnp
from jax import lax
from jax.experimental import pallas as pl
from jax.experimental.pallas import tpu as pltpu
from jax.experimental.pallas import tpu_sc as plsc

_NC = 2
_NS = 16
_NW = _NC * _NS

_LANE = 128
_SUB = 8
_G = _LANE * _SUB


@functools.cache
def _build_gather(B, D):
  assert B % (_NW * _G) == 0
  n_steps = B // (_NW * _G)
  b_per_w = B // _NW

  mesh = plsc.VectorSubcoreMesh(core_axis_name="c", subcore_axis_name="s")

  @functools.partial(
      pl.kernel,
      mesh=mesh,
      out_type=jax.ShapeDtypeStruct((B, D), jnp.float32),
      scratch_types=[
          pltpu.VMEM((n_steps * _SUB, _LANE), jnp.int32),
          pltpu.VMEM((_G, D), jnp.float32),
          pltpu.SemaphoreType.DMA,
      ],
      compiler_params=pltpu.CompilerParams(use_tc_tiling_on_sc=False),
  )
  def gather_kernel(idx_hbm, table_hbm, out_hbm, idx_v, rows_v, sem):
    wid = lax.axis_index("s") * _NC + lax.axis_index("c")
    pltpu.sync_copy(idx_hbm.at[wid], idx_v)

    def step(g, carry):
      copies = []
      for j in range(_SUB):
        copies.append(
            pltpu.async_copy(
                table_hbm.at[idx_v.at[g * _SUB + j]],
                rows_v.at[pl.ds(j * _LANE, _LANE)],
                sem,
            ))
      for c in copies:
        c.wait()
      pltpu.sync_copy(rows_v, out_hbm.at[pl.ds(wid * b_per_w + g * _G, _G)])
      return carry

    lax.fori_loop(0, n_steps, step, 0)

  return gather_kernel


def kernel(inputs, table):
  bt, s, n = inputs.shape
  b = bt * s * n
  d = table.shape[1]
  idx = inputs.reshape(_NW, b // (_NW * _LANE), _LANE).astype(jnp.int32)
  out = _build_gather(b, d)(idx, table)
  return out.reshape(bt, s, n, d)


# speedup vs baseline: 1.5817x; 1.0101x over previous
import functools

import jax
import jax.numpy as jnp
from jax import lax
from jax.experimental import pallas as pl
from jax.experimental.pallas import tpu as pltpu
from jax.experimental.pallas import tpu_sc as plsc

_NC = 2
_NS = 16
_NW = _NC * _NS

_LANE = 128
_SUB = 8
_G = _LANE * _SUB
_NBUF = 2


@functools.cache
def _build_gather(B, D):
  assert B % (_NW * _G) == 0
  n_steps = B // (_NW * _G)
  n_chunks = B // (_NW * _LANE)
  b_per_w = B // _NW

  mesh = plsc.VectorSubcoreMesh(core_axis_name="c", subcore_axis_name="s")

  @functools.partial(
      pl.kernel,
      mesh=mesh,
      out_type=jax.ShapeDtypeStruct((B, D), jnp.float32),
      scratch_types=[
          pltpu.VMEM((n_chunks, _LANE), jnp.int32),
          pltpu.VMEM((_NBUF, _G, D), jnp.float32),
          pltpu.SemaphoreType.DMA((_NBUF,)),
      ],
      compiler_params=pltpu.CompilerParams(use_tc_tiling_on_sc=False),
  )
  def gather_kernel(idx_hbm, table_hbm, out_hbm, idx_v, rows_v, gsems):
    wid = lax.axis_index("s") * _NC + lax.axis_index("c")
    base = wid * b_per_w
    pltpu.sync_copy(idx_hbm.at[wid], idx_v)

    def issue(g, b):
      for j in range(_SUB):
        pltpu.async_copy(
            table_hbm.at[idx_v.at[g * _SUB + j]],
            rows_v.at[b, pl.ds(j * _LANE, _LANE)],
            gsems.at[b],
        )

    for b in range(_NBUF):
      issue(b, b)

    def step(g, carry):
      b = lax.rem(g, _NBUF)
      pltpu.make_async_copy(
          table_hbm.at[pl.ds(0, _G)], rows_v.at[b], gsems.at[b]).wait()
      pltpu.sync_copy(rows_v.at[b], out_hbm.at[pl.ds(base + g * _G, _G)])

      @pl.when(g + _NBUF < n_steps)
      def _():
        issue(g + _NBUF, b)

      return carry

    lax.fori_loop(0, n_steps, step, 0)

  return gather_kernel


def kernel(inputs, table):
  bt, s, n = inputs.shape
  b = bt * s * n
  d = table.shape[1]
  idx = inputs.reshape(_NW, b // (_NW * _LANE), _LANE).astype(jnp.int32)
  out = _build_gather(b, d)(idx, table)
  return out.reshape(bt, s, n, d)
